# baseline (speedup 1.0000x reference)
import jax
import jax.numpy as jnp
from jax import lax
from jax.experimental import pallas as pl
from jax.experimental.pallas import tpu as pltpu
from jax.experimental.pallas import tpu_sc as plsc

_LEVEL = 15
_DIM = 2
_BASE_RES = 512
_N = 524288
_OUT_COLS = 1 + _LEVEL * _DIM

_OFFSETS = []
_SCALES = []
_off = 0
for _i in range(_LEVEL):
    _res = int(_BASE_RES * 2.0 ** _i)
    _OFFSETS.append(_off)
    _SCALES.append(float(_res))
    _off += _res + 8
_TOTAL_ROWS = _off

_NC, _NS = 2, 16
_NW = _NC * _NS
_PTS_PER_TILE = _N // _NW
_C = 1024
_CHUNKS = _PTS_PER_TILE // _C
_G = _C // 16
_IDX_N = 2 * _C
_DMA_IDX = 128
_NDMA = _IDX_N // _DMA_IDX

_TS_LEVELS = 6
_TS_ROWS = _OFFSETS[_TS_LEVELS]
_SP_LEVELS = 2
_SP_BASE = _TS_ROWS
_SP_ROWS = _OFFSETS[_TS_LEVELS + _SP_LEVELS] - _SP_BASE

_SPLIT_LEVEL = 10
_SPLIT_ROW = _OFFSETS[_SPLIT_LEVEL]
_B_ROWS = _TOTAL_ROWS - _SPLIT_ROW
_A_COLS = 1 + _DIM * _SPLIT_LEVEL
_B_COLS = _DIM * (_LEVEL - _SPLIT_LEVEL)


def _interp(buf_loads, pos):
    i0 = pos.astype(jnp.int32)
    w1 = pos - i0.astype(jnp.float32)
    w0 = 1.0 - w1
    v0d0, v1d0, v0d1, v1d1 = buf_loads(i0)
    return w0 * v0d0 + w1 * v1d0, w0 * v0d1 + w1 * v1d1


def _make_body(dma_levels, staged, row_base, out_cols, col_of, write_x):
    def body(x_hbm, d0_hbm, d1_hbm, out_hbm, *scratch):
        if staged:
            (x_v, idx_a, idx_b, vals_a, vals_b, out_v, st0_v, st1_v,
             sp0_v, sp1_v, sem_a, sem_b, sem_s) = scratch
        else:
            (x_v, idx_a, idx_b, vals_a, vals_b, out_v,
             sem_a, sem_b, sem_s) = scratch

        cid = lax.axis_index("c")
        sid = lax.axis_index("s")
        wid = cid * _NS + sid
        iota16 = lax.iota(jnp.int32, 16)

        if staged:
            ts_copies = [
                pltpu.async_copy(d0_hbm.at[pl.ds(0, _TS_ROWS)], st0_v, sem_s),
                pltpu.async_copy(d1_hbm.at[pl.ds(0, _TS_ROWS)], st1_v, sem_s),
            ]
            for cp in ts_copies:
                cp.wait()

            @pl.when(sid == 0)
            def _stage_spmem():
                sp_copies = [
                    pltpu.async_copy(
                        d0_hbm.at[pl.ds(_SP_BASE, _SP_ROWS)], sp0_v, sem_s),
                    pltpu.async_copy(
                        d1_hbm.at[pl.ds(_SP_BASE, _SP_ROWS)], sp1_v, sem_s),
                ]
                for cp in sp_copies:
                    cp.wait()

            plsc.subcore_barrier()

        vals = {0: vals_a, 1: vals_b}
        idxs = {0: idx_a, 1: idx_b}
        sems = {0: sem_a, 1: sem_b}

        def chunk_body(ci, carry):
            base = (wid * _CHUNKS + ci) * _C
            pltpu.sync_copy(x_hbm.at[pl.ds(base, _C)], x_v)

            def build_and_fire(l):
                scale = _SCALES[l]
                spmem = staged and l < _TS_LEVELS + _SP_LEVELS
                offl = _OFFSETS[l] - (_SP_BASE if spmem else row_base)
                idx_v = idxs[l % 2]

                def idx_body(g, c, scale=scale, offl=offl, idx_v=idx_v):
                    xv = x_v[pl.ds(g * 16, 16)]
                    pos = jnp.minimum(jnp.maximum(xv, 0.0), 1.0) * scale
                    i0 = pos.astype(jnp.int32) + offl
                    idx_v[pl.ds(g * 16, 16)] = i0
                    idx_v[pl.ds(_C + g * 16, 16)] = i0 + 1
                    return c

                lax.fori_loop(0, _G, idx_body, 0)

                s0 = sp0_v if spmem else d0_hbm
                s1 = sp1_v if spmem else d1_hbm
                buf, sem = vals[l % 2], sems[l % 2]
                copies = []
                for j in range(_NDMA):
                    isl = idx_v.at[pl.ds(j * _DMA_IDX, _DMA_IDX)]
                    copies.append(pltpu.async_copy(
                        s0.at[isl],
                        buf.at[pl.ds(j * _DMA_IDX, _DMA_IDX)], sem))
                    copies.append(pltpu.async_copy(
                        s1.at[isl],
                        buf.at[pl.ds(_IDX_N + j * _DMA_IDX, _DMA_IDX)], sem))
                return copies

            def comp_dma_level(l):
                scale = _SCALES[l]
                buf = vals[l % 2]
                col = col_of(l)

                def comp_body(g, c, scale=scale, col=col, buf=buf):
                    xv = x_v[pl.ds(g * 16, 16)]
                    pos = jnp.minimum(jnp.maximum(xv, 0.0), 1.0) * scale
                    s = g * 16

                    def loads(i0, s=s, buf=buf):
                        return (buf[pl.ds(s, 16)],
                                buf[pl.ds(_C + s, 16)],
                                buf[pl.ds(_IDX_N + s, 16)],
                                buf[pl.ds(_IDX_N + _C + s, 16)])

                    o0, o1 = _interp(loads, pos)
                    flat = (s + iota16) * out_cols + col
                    plsc.store_scatter(out_v, [flat], o0)
                    plsc.store_scatter(out_v, [flat + 1], o1)
                    return c

                lax.fori_loop(0, _G, comp_body, 0)

            inflight = build_and_fire(dma_levels[0])

            if write_x:
                def xcol_body(g, c):
                    xv = x_v[pl.ds(g * 16, 16)]
                    flat = (g * 16 + iota16) * out_cols
                    plsc.store_scatter(out_v, [flat], xv)
                    return c

                lax.fori_loop(0, _G, xcol_body, 0)

            if staged:
                for l in range(_TS_LEVELS):
                    scale = _SCALES[l]
                    offl = _OFFSETS[l]
                    col = col_of(l)

                    def comp_staged(g, c, scale=scale, offl=offl, col=col):
                        xv = x_v[pl.ds(g * 16, 16)]
                        pos = jnp.minimum(jnp.maximum(xv, 0.0), 1.0) * scale

                        def loads(i0, offl=offl):
                            r0 = i0 + offl
                            r1 = r0 + 1
                            return (plsc.load_gather(st0_v, [r0]),
                                    plsc.load_gather(st0_v, [r1]),
                                    plsc.load_gather(st1_v, [r0]),
                                    plsc.load_gather(st1_v, [r1]))

                        o0, o1 = _interp(loads, pos)
                        flat = (g * 16 + iota16) * out_cols + col
                        plsc.store_scatter(out_v, [flat], o0)
                        plsc.store_scatter(out_v, [flat + 1], o1)
                        return c

                    lax.fori_loop(0, _G, comp_staged, 0)

            for l in dma_levels[1:]:
                nxt = build_and_fire(l)
                for cp in inflight:
                    cp.wait()
                inflight = nxt
                comp_dma_level(l - 1)
            for cp in inflight:
                cp.wait()
            comp_dma_level(dma_levels[-1])

            pltpu.sync_copy(
                out_v, out_hbm.at[pl.ds(base * out_cols, _C * out_cols)])
            return carry

        lax.fori_loop(0, _CHUNKS, chunk_body, 0)

    return body


def _make_call(dma_levels, staged, row_base, out_cols, col_of, write_x):
    scratch = [
        pltpu.VMEM((_C,), jnp.float32),
        pltpu.VMEM((_IDX_N,), jnp.int32),
        pltpu.VMEM((_IDX_N,), jnp.int32),
        pltpu.VMEM((2 * _IDX_N,), jnp.float32),
        pltpu.VMEM((2 * _IDX_N,), jnp.float32),
        pltpu.VMEM((_C * out_cols,), jnp.float32),
    ]
    if staged:
        scratch += [
            pltpu.VMEM((_TS_ROWS,), jnp.float32),
            pltpu.VMEM((_TS_ROWS,), jnp.float32),
            pltpu.VMEM_SHARED((_SP_ROWS,), jnp.float32),
            pltpu.VMEM_SHARED((_SP_ROWS,), jnp.float32),
        ]
    scratch += [pltpu.SemaphoreType.DMA] * 3
    return pl.kernel(
        _make_body(dma_levels, staged, row_base, out_cols, col_of, write_x),
        out_type=jax.ShapeDtypeStruct((_N * out_cols,), jnp.float32),
        mesh=plsc.VectorSubcoreMesh(core_axis_name="c", subcore_axis_name="s"),
        compiler_params=pltpu.CompilerParams(
            needs_layout_passes=False, use_tc_tiling_on_sc=False
        ),
        scratch_types=scratch,
    )


_call_a = _make_call(
    dma_levels=list(range(_TS_LEVELS + _SP_LEVELS, _SPLIT_LEVEL)),
    staged=True, row_base=0, out_cols=_A_COLS,
    col_of=lambda l: 1 + _DIM * l, write_x=True,
)
_call_b = _make_call(
    dma_levels=list(range(_SPLIT_LEVEL, _LEVEL)),
    staged=False, row_base=_SPLIT_ROW, out_cols=_B_COLS,
    col_of=lambda l: _DIM * (l - _SPLIT_LEVEL), write_x=False,
)


def kernel(x, data):
    xf = x.reshape(-1)
    dt = data.T
    dta = lax.optimization_barrier(dt)
    a0 = dta[0, :_SPLIT_ROW]
    a1 = dta[1, :_SPLIT_ROW]
    b0 = dt[0, _SPLIT_ROW:]
    b1 = dt[1, _SPLIT_ROW:]
    out_a = _call_a(xf, a0, a1).reshape(_N, _A_COLS)
    out_b = _call_b(xf, b0, b1).reshape(_N, _B_COLS)
    return jnp.concatenate([out_a, out_b], axis=1)


# speedup vs baseline: 1.2912x; 1.2912x over previous
import jax
import jax.numpy as jnp
from jax import lax
from jax.experimental import pallas as pl
from jax.experimental.pallas import tpu as pltpu
from jax.experimental.pallas import tpu_sc as plsc

_LEVEL = 15
_DIM = 2
_BASE_RES = 512
_N = 524288
_OUT_COLS = 1 + _LEVEL * _DIM

_OFFSETS = []
_SCALES = []
_off = 0
for _i in range(_LEVEL):
    _res = int(_BASE_RES * 2.0 ** _i)
    _OFFSETS.append(_off)
    _SCALES.append(float(_res))
    _off += _res + 8
_TOTAL_ROWS = _off

_NC, _NS = 2, 16
_NW = _NC * _NS
_PTS_PER_TILE = _N // _NW
_C = 1024
_CHUNKS = _PTS_PER_TILE // _C
_G = _C // 16
_IDX_N = 2 * _C
_DMA_IDX = 128
_NDMA = _IDX_N // _DMA_IDX

_TS_LEVELS = 6
_TS_ROWS = _OFFSETS[_TS_LEVELS]
_SP_LEVELS = 2
_SP_BASE = _TS_ROWS
_SP_ROWS = _OFFSETS[_TS_LEVELS + _SP_LEVELS] - _SP_BASE
_DMA_LEVELS = list(range(_TS_LEVELS, _LEVEL))


def _mrl_body(
    x_hbm, d0_hbm, d1_hbm, out_hbm,
    x_v, idx_a, idx_b, vals_a, vals_b, out_v, st0_v, st1_v, sp0_v, sp1_v,
    sem_a, sem_b, sem_s,
):
    cid = lax.axis_index("c")
    sid = lax.axis_index("s")
    wid = cid * _NS + sid
    iota16 = lax.iota(jnp.int32, 16)

    ts_copies = [
        pltpu.async_copy(d0_hbm.at[pl.ds(0, _TS_ROWS)], st0_v, sem_s),
        pltpu.async_copy(d1_hbm.at[pl.ds(0, _TS_ROWS)], st1_v, sem_s),
    ]
    for cp in ts_copies:
        cp.wait()

    @pl.when(sid == 0)
    def _stage_spmem():
        sp_copies = [
            pltpu.async_copy(d0_hbm.at[pl.ds(_SP_BASE, _SP_ROWS)], sp0_v, sem_s),
            pltpu.async_copy(d1_hbm.at[pl.ds(_SP_BASE, _SP_ROWS)], sp1_v, sem_s),
        ]
        for cp in sp_copies:
            cp.wait()

    plsc.subcore_barrier()

    vals = {0: vals_a, 1: vals_b}
    idxs = {0: idx_a, 1: idx_b}
    sems = {0: sem_a, 1: sem_b}

    def chunk_body(ci, carry):
        base = (wid * _CHUNKS + ci) * _C
        pltpu.sync_copy(x_hbm.at[pl.ds(base, _C)], x_v)

        def build_and_fire(l):
            scale = _SCALES[l]
            spmem = l < _TS_LEVELS + _SP_LEVELS
            offl = _OFFSETS[l] - (_SP_BASE if spmem else 0)
            idx_v = idxs[l % 2]

            def idx_body(g, c, scale=scale, offl=offl, idx_v=idx_v):
                xv = x_v[pl.ds(g * 16, 16)]
                pos = jnp.minimum(jnp.maximum(xv, 0.0), 1.0) * scale
                i0 = pos.astype(jnp.int32) + offl
                r2 = (g * 16 + iota16) * 2
                plsc.store_scatter(idx_v, [r2], i0)
                plsc.store_scatter(idx_v, [r2 + 1], i0 + 1)
                return c

            lax.fori_loop(0, _G, idx_body, 0)

            s0 = sp0_v if spmem else d0_hbm
            s1 = sp1_v if spmem else d1_hbm
            buf, sem = vals[l % 2], sems[l % 2]
            copies = []
            for j in range(_NDMA):
                isl = idx_v.at[pl.ds(j * _DMA_IDX, _DMA_IDX)]
                copies.append(
                    pltpu.async_copy(
                        s0.at[isl],
                        buf.at[pl.ds(j * _DMA_IDX, _DMA_IDX)],
                        sem,
                    )
                )
                copies.append(
                    pltpu.async_copy(
                        s1.at[isl],
                        buf.at[pl.ds(_IDX_N + j * _DMA_IDX, _DMA_IDX)],
                        sem,
                    )
                )
            return copies

        def comp_dma_level(l):
            scale = _SCALES[l]
            buf = vals[l % 2]

            def comp_body(g, c, scale=scale, l=l, buf=buf):
                xv = x_v[pl.ds(g * 16, 16)]
                pos = jnp.minimum(jnp.maximum(xv, 0.0), 1.0) * scale
                i0 = pos.astype(jnp.int32)
                w1 = pos - i0.astype(jnp.float32)
                w0 = 1.0 - w1
                s = g * 16
                p2 = (s + iota16) * 2
                v0d0 = plsc.load_gather(buf, [p2])
                v1d0 = plsc.load_gather(buf, [p2 + 1])
                v0d1 = plsc.load_gather(buf, [p2 + _IDX_N])
                v1d1 = plsc.load_gather(buf, [p2 + _IDX_N + 1])
                o0 = w0 * v0d0 + w1 * v1d0
                o1 = w0 * v0d1 + w1 * v1d1
                flat = (s + iota16) * _OUT_COLS + (1 + _DIM * l)
                plsc.store_scatter(out_v, [flat], o0)
                plsc.store_scatter(out_v, [flat + 1], o1)
                return c

            lax.fori_loop(0, _G, comp_body, 0)

        inflight = build_and_fire(_DMA_LEVELS[0])

        def xcol_body(g, c):
            xv = x_v[pl.ds(g * 16, 16)]
            flat = (g * 16 + iota16) * _OUT_COLS
            plsc.store_scatter(out_v, [flat], xv)
            return c

        lax.fori_loop(0, _G, xcol_body, 0)

        for l in range(_TS_LEVELS):
            scale = _SCALES[l]
            offl = _OFFSETS[l]

            def comp_staged(g, c, scale=scale, offl=offl, l=l):
                xv = x_v[pl.ds(g * 16, 16)]
                pos = jnp.minimum(jnp.maximum(xv, 0.0), 1.0) * scale
                i0 = pos.astype(jnp.int32)
                w1 = pos - i0.astype(jnp.float32)
                w0 = 1.0 - w1
                r0 = i0 + offl
                r1 = r0 + 1
                v0d0 = plsc.load_gather(st0_v, [r0])
                v1d0 = plsc.load_gather(st0_v, [r1])
                v0d1 = plsc.load_gather(st1_v, [r0])
                v1d1 = plsc.load_gather(st1_v, [r1])
                o0 = w0 * v0d0 + w1 * v1d0
                o1 = w0 * v0d1 + w1 * v1d1
                flat = (g * 16 + iota16) * _OUT_COLS + (1 + _DIM * l)
                plsc.store_scatter(out_v, [flat], o0)
                plsc.store_scatter(out_v, [flat + 1], o1)
                return c

            lax.fori_loop(0, _G, comp_staged, 0)

        for l in _DMA_LEVELS[1:]:
            nxt = build_and_fire(l)
            for cp in inflight:
                cp.wait()
            inflight = nxt
            comp_dma_level(l - 1)
        for cp in inflight:
            cp.wait()
        comp_dma_level(_DMA_LEVELS[-1])

        pltpu.sync_copy(out_v, out_hbm.at[pl.ds(base * _OUT_COLS, _C * _OUT_COLS)])
        return carry

    lax.fori_loop(0, _CHUNKS, chunk_body, 0)


_mrl_call = pl.kernel(
    _mrl_body,
    out_type=jax.ShapeDtypeStruct((_N * _OUT_COLS,), jnp.float32),
    mesh=plsc.VectorSubcoreMesh(core_axis_name="c", subcore_axis_name="s"),
    compiler_params=pltpu.CompilerParams(
        needs_layout_passes=False, use_tc_tiling_on_sc=False
    ),
    scratch_types=[
        pltpu.VMEM((_C,), jnp.float32),
        pltpu.VMEM((_IDX_N,), jnp.int32),
        pltpu.VMEM((_IDX_N,), jnp.int32),
        pltpu.VMEM((2 * _IDX_N,), jnp.float32),
        pltpu.VMEM((2 * _IDX_N,), jnp.float32),
        pltpu.VMEM((_C * _OUT_COLS,), jnp.float32),
        pltpu.VMEM((_TS_ROWS,), jnp.float32),
        pltpu.VMEM((_TS_ROWS,), jnp.float32),
        pltpu.VMEM_SHARED((_SP_ROWS,), jnp.float32),
        pltpu.VMEM_SHARED((_SP_ROWS,), jnp.float32),
        pltpu.SemaphoreType.DMA,
        pltpu.SemaphoreType.DMA,
        pltpu.SemaphoreType.DMA,
    ],
)


def kernel(x, data):
    dt = data.T
    out = _mrl_call(x.reshape(-1), dt[0], dt[1])
    return out.reshape(_N, _OUT_COLS)
